# baseline (speedup 1.0000x reference)
import jax
import jax.numpy as jnp
from jax import lax
from jax.experimental import pallas as pl
from jax.experimental.pallas import tpu as pltpu
from jax.experimental.pallas import tpu_sc as plsc

NC = 2
NS = 16
NW = NC * NS
LANES = 16

B = 4096
L = 200
D = 64
SEQ_PER_W = B // NW
SUBSLICES = ((0, 80), (80, 80), (160, 40))
SHIFT_OFFS = tuple(range(0, 192, 16)) + (184,)


def _body(x_hbm, tok_hbm, pos_hbm, out_hbm,
          idx_res, pos_v, pidx_v, rows_v, out_v, gsem, osem):
    cid = lax.axis_index("c")
    sid = lax.axis_index("s")
    wid = sid * NC + cid
    seq_base = wid * SEQ_PER_W

    pltpu.sync_copy(x_hbm.at[pl.ds(seq_base, SEQ_PER_W)], idx_res)
    pltpu.sync_copy(pos_hbm, pos_v)

    def seq_body(c, carry):
        for o in SHIFT_OFFS:
            pidx_v[pl.ds(o, LANES)] = lax.shift_right_logical(
                idx_res[c, pl.ds(o, LANES)], 1
            )

        copies = []
        for o, w in SUBSLICES:
            copies.append(
                pltpu.async_copy(
                    tok_hbm.at[pidx_v.at[pl.ds(o, w)]],
                    rows_v.at[pl.ds(o, w)],
                    gsem,
                )
            )
        for cp in copies:
            cp.wait()

        def group_body(o, carry2):
            hvec = (idx_res[c, pl.ds(o, LANES)] & 1) * D
            for s in range(LANES):
                r = o + s
                h = hvec[s]
                for j in range(D // LANES):
                    out_v[r, pl.ds(j * LANES, LANES)] = (
                        rows_v[r, pl.ds(h + j * LANES, LANES)]
                        + pos_v[r, pl.ds(j * LANES, LANES)]
                    )
            return carry2

        lax.fori_loop(0, L // LANES, lambda g, c2: group_body(g * LANES, c2), 0)
        group_body(L - LANES, 0)

        pltpu.sync_copy(out_v, out_hbm.at[seq_base + c])
        return carry

    lax.fori_loop(0, SEQ_PER_W, seq_body, 0)


@jax.jit
def kernel(x, token_table, pos_table):
    mesh = plsc.VectorSubcoreMesh(core_axis_name="c", subcore_axis_name="s")
    tok_pairs = token_table.reshape(token_table.shape[0] // 2, 2 * D)
    out = pl.kernel(
        _body,
        mesh=mesh,
        out_type=jax.ShapeDtypeStruct((B, L, D), jnp.float32),
        compiler_params=pltpu.CompilerParams(use_tc_tiling_on_sc=True),
        scratch_types=[
            pltpu.VMEM((SEQ_PER_W, L), jnp.int32),
            pltpu.VMEM((L, D), jnp.float32),
            pltpu.VMEM((L,), jnp.int32),
            pltpu.VMEM((L, 2 * D), jnp.float32),
            pltpu.VMEM((L, D), jnp.float32),
            pltpu.SemaphoreType.DMA,
            pltpu.SemaphoreType.DMA,
        ],
    )(x.astype(jnp.int32), tok_pairs, pos_table)
    return out


# speedup vs baseline: 1.0210x; 1.0210x over previous
import jax
import jax.numpy as jnp
from jax import lax
from jax.experimental import pallas as pl
from jax.experimental.pallas import tpu as pltpu
from jax.experimental.pallas import tpu_sc as plsc

NC = 2
NS = 16
NW = NC * NS
LANES = 16

B = 4096
L = 200
D = 64
V = 100000
TILEB = B // NW
NBLK = L // 8


def _body(xT_hbm, tok_hbm, pos_hbm, out_hbm,
          idx_v, rows0, rows1, outT0, outT1, pos_v,
          gsem0, gsem1, osem0, osem1):
    cid = lax.axis_index("c")
    sid = lax.axis_index("s")
    wid = sid * NC + cid
    bofs = wid * TILEB

    rows = (rows0, rows1)
    outT = (outT0, outT1)
    gsem = (gsem0, gsem1)
    osem = (osem0, osem1)

    pltpu.sync_copy(pos_hbm, pos_v)
    pltpu.sync_copy(xT_hbm.at[pl.ds(0, 8), pl.ds(bofs, TILEB)], idx_v.at[0])

    def fire_gather(tile, k, buf):
        return pltpu.async_copy(
            tok_hbm.at[idx_v.at[tile, k]], rows[buf], gsem[buf]
        )

    def wait_gather(buf):
        pltpu.make_async_copy(
            tok_hbm.at[idx_v.at[0, 0]], rows[buf], gsem[buf]
        ).wait()

    def wait_out(buf):
        pltpu.make_async_copy(
            outT[buf], out_hbm.at[0, :, pl.ds(bofs, TILEB)], osem[buf]
        ).wait()

    def transpose_add(l, buf):
        pv = [pos_v[l, pl.ds(16 * d16, LANES)] for d16 in range(D // LANES)]
        ii = lax.iota(jnp.int32, LANES)

        def jbody(j, car):
            jj = jnp.full((LANES,), j, jnp.int32)
            for d16 in range(D // LANES):
                val = rows[buf][j, pl.ds(16 * d16, LANES)] + pv[d16]
                plsc.store_scatter(outT[buf], [16 * d16 + ii, jj], val)
            return car

        lax.fori_loop(0, TILEB, jbody, 0)

    def fire_out(l, buf):
        return pltpu.async_copy(
            outT[buf], out_hbm.at[l, :, pl.ds(bofs, TILEB)], osem[buf]
        )

    fire_gather(0, 0, 0)

    def step(t, k, first_block):
        l = 8 * t + k
        p = k & 1
        cur = t % 2
        nxt = (t + 1) % 2
        if k < 7:
            fire_gather(cur, k + 1, p ^ 1)
        else:
            fire_gather(nxt, 0, p ^ 1)
        wait_gather(p)
        if not (first_block and l < 2):
            wait_out(p)
        transpose_add(l, p)
        fire_out(l, p)

    pltpu.sync_copy(xT_hbm.at[pl.ds(8, 8), pl.ds(bofs, TILEB)], idx_v.at[1])
    for k in range(8):
        step(0, k, True)

    def block_body(t, car):
        src = jnp.minimum((t + 1) * 8, L - 8)
        pltpu.sync_copy(
            xT_hbm.at[pl.ds(src, 8), pl.ds(bofs, TILEB)],
            idx_v.at[(t + 1) % 2],
        )
        for k in range(8):
            step(t, k, False)
        return car

    lax.fori_loop(1, NBLK, block_body, 0)

    wait_gather(0)
    wait_out(0)
    wait_out(1)


@jax.jit
def kernel(x, token_table, pos_table):
    mesh = plsc.VectorSubcoreMesh(core_axis_name="c", subcore_axis_name="s")
    xT = x.T.astype(jnp.int32)
    tok_pad = jnp.pad(token_table, ((0, 0), (0, 2 * D - token_table.shape[1])))
    outT = pl.kernel(
        _body,
        mesh=mesh,
        out_type=jax.ShapeDtypeStruct((L, D, B), jnp.float32),
        compiler_params=pltpu.CompilerParams(
            use_tc_tiling_on_sc=True, needs_layout_passes=False
        ),
        scratch_types=[
            pltpu.VMEM((2, 8, TILEB), jnp.int32),
            pltpu.VMEM((TILEB, 2 * D), jnp.float32),
            pltpu.VMEM((TILEB, 2 * D), jnp.float32),
            pltpu.VMEM((D, TILEB), jnp.float32),
            pltpu.VMEM((D, TILEB), jnp.float32),
            pltpu.VMEM((L, D), jnp.float32),
            pltpu.SemaphoreType.DMA,
            pltpu.SemaphoreType.DMA,
            pltpu.SemaphoreType.DMA,
            pltpu.SemaphoreType.DMA,
        ],
    )(xT, tok_pad, pos_table)
    return outT.transpose(2, 0, 1)


# speedup vs baseline: 1.5740x; 1.5416x over previous
import jax
import jax.numpy as jnp
from jax import lax
from jax.experimental import pallas as pl
from jax.experimental.pallas import tpu as pltpu
from jax.experimental.pallas import tpu_sc as plsc

NC = 2
NS = 16
NW = NC * NS
LANES = 16

B = 4096
L = 200
D = 64
V = 100000
TILEB = B // NW
NBLK = L // 8


def _body(xT_hbm, tok_hbm, pos_hbm, out_hbm,
          idx_v, rows0, rows1, outT0, outT1, pos_v,
          gsem0, gsem1, osem0, osem1):
    cid = lax.axis_index("c")
    sid = lax.axis_index("s")
    wid = sid * NC + cid
    bofs = wid * TILEB

    rows = (rows0, rows1)
    outT = (outT0, outT1)
    gsem = (gsem0, gsem1)
    osem = (osem0, osem1)

    pltpu.sync_copy(pos_hbm, pos_v)
    pltpu.sync_copy(xT_hbm.at[pl.ds(0, 8), pl.ds(bofs, TILEB)], idx_v.at[0])

    def fire_gather(tile, k, buf):
        return pltpu.async_copy(
            tok_hbm.at[idx_v.at[tile, k]], rows[buf], gsem[buf]
        )

    def wait_gather(buf):
        pltpu.make_async_copy(
            tok_hbm.at[idx_v.at[0, 0]], rows[buf], gsem[buf]
        ).wait()

    def wait_out(buf):
        pltpu.make_async_copy(
            outT[buf], out_hbm.at[0, :, pl.ds(bofs, TILEB)], osem[buf]
        ).wait()

    def transpose_add(l, buf):
        pv = [pos_v[l, pl.ds(16 * d16, LANES)] for d16 in range(D // LANES)]
        ii = lax.iota(jnp.int32, LANES)

        @plsc.parallel_loop(0, TILEB, step=1, unroll=8)
        def jbody(j):
            jj = jnp.full((LANES,), j, jnp.int32)
            for d16 in range(D // LANES):
                val = rows[buf][j, pl.ds(16 * d16, LANES)] + pv[d16]
                plsc.store_scatter(outT[buf], [16 * d16 + ii, jj], val)

    def fire_out(l, buf):
        return pltpu.async_copy(
            outT[buf], out_hbm.at[l, :, pl.ds(bofs, TILEB)], osem[buf]
        )

    fire_gather(0, 0, 0)

    def step(t, k, first_block):
        l = 8 * t + k
        p = k & 1
        cur = t % 2
        nxt = (t + 1) % 2
        if k < 7:
            fire_gather(cur, k + 1, p ^ 1)
        else:
            fire_gather(nxt, 0, p ^ 1)
        wait_gather(p)
        if not (first_block and l < 2):
            wait_out(p)
        transpose_add(l, p)
        fire_out(l, p)

    pltpu.sync_copy(xT_hbm.at[pl.ds(8, 8), pl.ds(bofs, TILEB)], idx_v.at[1])
    for k in range(8):
        step(0, k, True)

    def block_body(t, car):
        src = jnp.minimum((t + 1) * 8, L - 8)
        pltpu.sync_copy(
            xT_hbm.at[pl.ds(src, 8), pl.ds(bofs, TILEB)],
            idx_v.at[(t + 1) % 2],
        )
        for k in range(8):
            step(t, k, False)
        return car

    lax.fori_loop(1, NBLK, block_body, 0)

    wait_gather(0)
    wait_out(0)
    wait_out(1)


@jax.jit
def kernel(x, token_table, pos_table):
    mesh = plsc.VectorSubcoreMesh(core_axis_name="c", subcore_axis_name="s")
    xT = x.T.astype(jnp.int32)
    tok_pad = jnp.pad(token_table, ((0, 0), (0, 2 * D - token_table.shape[1])))
    outT = pl.kernel(
        _body,
        mesh=mesh,
        out_type=jax.ShapeDtypeStruct((L, D, B), jnp.float32),
        compiler_params=pltpu.CompilerParams(
            use_tc_tiling_on_sc=True, needs_layout_passes=False
        ),
        scratch_types=[
            pltpu.VMEM((2, 8, TILEB), jnp.int32),
            pltpu.VMEM((TILEB, 2 * D), jnp.float32),
            pltpu.VMEM((TILEB, 2 * D), jnp.float32),
            pltpu.VMEM((D, TILEB), jnp.float32),
            pltpu.VMEM((D, TILEB), jnp.float32),
            pltpu.VMEM((L, D), jnp.float32),
            pltpu.SemaphoreType.DMA,
            pltpu.SemaphoreType.DMA,
            pltpu.SemaphoreType.DMA,
            pltpu.SemaphoreType.DMA,
        ],
    )(xT, tok_pad, pos_table)
    return outT.transpose(2, 0, 1)
